# Initial kernel scaffold; baseline (speedup 1.0000x reference)
import jax
import jax.numpy as jnp
from jax.experimental import pallas as pl


def kernel(x, moneyness_bins, maturity_bins, W1, b1, W2, b2, W3, b3):
    raise NotImplementedError("write your pallas kernel here")


# speedup vs baseline: 4.3913x; 4.3913x over previous
import functools

import jax
import jax.numpy as jnp
from jax import lax
from jax.experimental import pallas as pl
from jax.experimental.pallas import tpu as pltpu
from jax.experimental.pallas import tpu_sc as plsc

NEXP = 16
NTOK = 16384
DIN = 23
DPAD = 128
H = 512
BLK = 256
P_CAP = NTOK + NEXP * BLK
NB = P_CAP // BLK
R = 128
C = 128
NW = 32
TPW = NTOK // NW
CH = TPW // 128


def _routing_body(mb_ref, tb_ref, dest_ref, be_ref):
    flat = mb_ref[...] * 4 + tb_ref[...]
    ri = lax.broadcasted_iota(jnp.int32, (R, C), 0)
    ci = lax.broadcasted_iota(jnp.int32, (R, C), 1)
    upper = (ri <= ci).astype(jnp.float32)
    lower = (ci < ri).astype(jnp.float32)
    ones = jnp.ones((R, C), jnp.float32)

    rank = jnp.zeros((R, C), jnp.float32)
    counts = []
    for e in range(NEXP):
        m = (flat == e).astype(jnp.float32)
        incl = jnp.dot(m, upper, preferred_element_type=jnp.float32)
        col_pre = jnp.dot(lower, m, preferred_element_type=jnp.float32)
        row_pre = jnp.dot(col_pre, ones, preferred_element_type=jnp.float32)
        rank = rank + m * (incl - m + row_pre)
        counts.append(jnp.sum(m).astype(jnp.int32))

    offs = []
    cumnb = []
    acc = jnp.int32(0)
    for e in range(NEXP):
        nbe = (counts[e] + (BLK - 1)) // BLK
        offs.append(acc * BLK)
        acc = acc + nbe
        cumnb.append(acc)

    offvec = jnp.zeros((R, C), jnp.float32)
    for e in range(NEXP):
        m = (flat == e).astype(jnp.float32)
        offvec = offvec + m * offs[e].astype(jnp.float32)
    dest_ref[...] = (rank + offvec).astype(jnp.int32)

    k = (lax.broadcasted_iota(jnp.int32, (8, 128), 0) * 128
         + lax.broadcasted_iota(jnp.int32, (8, 128), 1))
    be = jnp.zeros((8, 128), jnp.int32)
    for e in range(NEXP):
        be = be + (k >= cumnb[e]).astype(jnp.int32)
    be_ref[...] = jnp.minimum(be, NEXP - 1)


def _routing_call(mb, tb, interpret=False):
    return pl.pallas_call(
        _routing_body,
        out_shape=(jax.ShapeDtypeStruct((R, C), jnp.int32),
                   jax.ShapeDtypeStruct((8, 128), jnp.int32)),
        interpret=interpret,
    )(mb, tb)


def _mlp_body(be_ref, x_ref, w1_ref, b1_ref, w2_ref, b2_ref, w3_ref, b3_ref,
              o_ref):
    x = x_ref[...]
    h = jnp.dot(x.astype(jnp.bfloat16), w1_ref[0],
                preferred_element_type=jnp.float32)
    h = jnp.maximum(h + b1_ref[0], 0.0)
    h2 = jnp.dot(h.astype(jnp.bfloat16), w2_ref[0],
                 preferred_element_type=jnp.float32)
    h2 = jnp.maximum(h2 + b2_ref[0], 0.0)
    y = jnp.sum(h2 * w3_ref[0], axis=1, keepdims=True) + b3_ref[0, 0, 0]
    o_ref[...] = jnp.broadcast_to(y, (BLK, DPAD))


def _mlp_call(be, xpad, w1p, b1, w2bf, b2, w3r, b3, interpret=False):
    grid_spec = pltpu.PrefetchScalarGridSpec(
        num_scalar_prefetch=1,
        grid=(NB,),
        in_specs=[
            pl.BlockSpec((BLK, DPAD), lambda i, be: (i, 0)),
            pl.BlockSpec((1, DPAD, H), lambda i, be: (be[i], 0, 0)),
            pl.BlockSpec((1, 1, H), lambda i, be: (be[i], 0, 0)),
            pl.BlockSpec((1, H, H), lambda i, be: (be[i], 0, 0)),
            pl.BlockSpec((1, 1, H), lambda i, be: (be[i], 0, 0)),
            pl.BlockSpec((1, 1, H), lambda i, be: (be[i], 0, 0)),
            pl.BlockSpec((1, 1, 1), lambda i, be: (be[i], 0, 0)),
        ],
        out_specs=pl.BlockSpec((BLK, DPAD), lambda i, be: (i, 0)),
    )
    return pl.pallas_call(
        _mlp_body,
        grid_spec=grid_spec,
        out_shape=jax.ShapeDtypeStruct((P_CAP, DPAD), jnp.float32),
        interpret=interpret,
    )(be, xpad, w1p, b1, w2bf, b2, w3r, b3)


@functools.lru_cache(maxsize=1)
def _sc_kernels():
    mesh = plsc.VectorSubcoreMesh(core_axis_name="c", subcore_axis_name="s")

    @functools.partial(
        pl.kernel,
        mesh=mesh,
        out_type=jax.ShapeDtypeStruct((P_CAP, DPAD), jnp.float32),
        scratch_types=[
            pltpu.VMEM((CH, 128), jnp.int32),
            pltpu.VMEM((TPW, DPAD), jnp.float32),
            pltpu.SemaphoreType.DMA,
        ],
    )
    def _sc_scatter(dest_hbm, x_hbm, xpad_hbm, idx_v, rows_v, sem):
        wid = lax.axis_index("s") * 2 + lax.axis_index("c")
        pltpu.sync_copy(dest_hbm.at[wid], idx_v)
        pltpu.sync_copy(x_hbm.at[pl.ds(wid * TPW, TPW)], rows_v)
        for j in range(CH):
            pltpu.async_copy(rows_v.at[pl.ds(j * 128, 128)],
                             xpad_hbm.at[idx_v.at[j]], sem).wait()

    @functools.partial(
        pl.kernel,
        mesh=mesh,
        out_type=jax.ShapeDtypeStruct((NTOK, DPAD), jnp.float32),
        scratch_types=[
            pltpu.VMEM((CH, 128), jnp.int32),
            pltpu.VMEM((TPW, DPAD), jnp.float32),
            pltpu.SemaphoreType.DMA,
        ],
    )
    def _sc_gather(dest_hbm, ypad_hbm, out_hbm, idx_v, rows_v, sem):
        wid = lax.axis_index("s") * 2 + lax.axis_index("c")
        pltpu.sync_copy(dest_hbm.at[wid], idx_v)
        for j in range(CH):
            pltpu.async_copy(ypad_hbm.at[idx_v.at[j]],
                             rows_v.at[pl.ds(j * 128, 128)], sem).wait()
        pltpu.sync_copy(rows_v, out_hbm.at[pl.ds(wid * TPW, TPW)])

    return _sc_scatter, _sc_gather


def kernel(x, moneyness_bins, maturity_bins, W1, b1, W2, b2, W3, b3):
    mb = moneyness_bins.astype(jnp.int32).reshape(R, C)
    tb = maturity_bins.astype(jnp.int32).reshape(R, C)
    dest, be_full = _routing_call(mb, tb)
    be = be_full.reshape(-1)[:NB]
    dest3 = dest.reshape(NW, CH, 128)

    sc_scatter, sc_gather = _sc_kernels()
    xp = jnp.pad(x, ((0, 0), (0, DPAD - DIN)))
    xpad = sc_scatter(dest3, xp)

    w1p = jnp.pad(W1, ((0, 0), (0, DPAD - DIN), (0, 0))).astype(jnp.bfloat16)
    ypad = _mlp_call(be, xpad, w1p, b1.reshape(NEXP, 1, H),
                     W2.astype(jnp.bfloat16), b2.reshape(NEXP, 1, H),
                     W3.reshape(NEXP, 1, H), b3.reshape(NEXP, 1, 1))

    outr = sc_gather(dest3, ypad)
    return outr[:, :1]


# speedup vs baseline: 5.3356x; 1.2151x over previous
import functools

import jax
import jax.numpy as jnp
from jax import lax
from jax.experimental import pallas as pl
from jax.experimental.pallas import tpu as pltpu
from jax.experimental.pallas import tpu_sc as plsc

NEXP = 16
NTOK = 16384
DIN = 23
DPAD = 128
H = 512
BLK = 512
P_CAP = NTOK + NEXP * BLK
NB = P_CAP // BLK
R = 128
C = 128
NW = 32
TPW = NTOK // NW
CH = TPW // 128


def _routing_body(mb_ref, tb_ref, dest_ref, be_ref):
    flat = mb_ref[...] * 4 + tb_ref[...]
    ri = lax.broadcasted_iota(jnp.int32, (R, C), 0)
    ci = lax.broadcasted_iota(jnp.int32, (R, C), 1)
    upper = (ri <= ci).astype(jnp.float32)
    lower = (ci < ri).astype(jnp.float32)
    ones = jnp.ones((R, C), jnp.float32)

    rank = jnp.zeros((R, C), jnp.float32)
    counts = []
    for e in range(NEXP):
        m = (flat == e).astype(jnp.float32)
        incl = jnp.dot(m, upper, preferred_element_type=jnp.float32)
        col_pre = jnp.dot(lower, m, preferred_element_type=jnp.float32)
        row_pre = jnp.dot(col_pre, ones, preferred_element_type=jnp.float32)
        rank = rank + m * (incl - m + row_pre)
        counts.append(jnp.sum(m).astype(jnp.int32))

    offs = []
    cumnb = []
    acc = jnp.int32(0)
    for e in range(NEXP):
        nbe = (counts[e] + (BLK - 1)) // BLK
        offs.append(acc * BLK)
        acc = acc + nbe
        cumnb.append(acc)

    offvec = jnp.zeros((R, C), jnp.float32)
    for e in range(NEXP):
        m = (flat == e).astype(jnp.float32)
        offvec = offvec + m * offs[e].astype(jnp.float32)
    dest_ref[...] = (rank + offvec).astype(jnp.int32)

    k = (lax.broadcasted_iota(jnp.int32, (8, 128), 0) * 128
         + lax.broadcasted_iota(jnp.int32, (8, 128), 1))
    be = jnp.zeros((8, 128), jnp.int32)
    for e in range(NEXP):
        be = be + (k >= cumnb[e]).astype(jnp.int32)
    be = jnp.minimum(be, NEXP - 1)
    be_ref[...] = jnp.where(k == NB, cumnb[NEXP - 1], be)


def _routing_call(mb, tb, interpret=False):
    return pl.pallas_call(
        _routing_body,
        out_shape=(jax.ShapeDtypeStruct((R, C), jnp.int32),
                   jax.ShapeDtypeStruct((8, 128), jnp.int32)),
        interpret=interpret,
    )(mb, tb)


def _mlp_body(be_ref, used_ref, x_ref, w1_ref, b1_ref, w2_ref, b2_ref,
              w3_ref, b3_ref, o_ref):
    @pl.when(pl.program_id(0) < used_ref[0])
    def _():
        x = x_ref[...]
        h = jnp.dot(x.astype(jnp.bfloat16), w1_ref[0],
                    preferred_element_type=jnp.float32)
        h = jnp.maximum(h + b1_ref[0], 0.0)
        h2 = jnp.dot(h.astype(jnp.bfloat16), w2_ref[0],
                     preferred_element_type=jnp.float32)
        h2 = jnp.maximum(h2 + b2_ref[0], 0.0)
        y = (jnp.sum(h2 * w3_ref[0], axis=1, keepdims=True)
             + b3_ref[0, 0, 0])
        o_ref[...] = jnp.broadcast_to(y, (BLK, DPAD))


def _mlp_call(be, used, xpad, w1p, b1, w2bf, b2, w3r, b3, interpret=False):
    def _blk(i, be, used):
        return jnp.minimum(i, used[0] - 1)

    def _exp(i, be, used):
        return be[jnp.minimum(i, used[0] - 1)]

    grid_spec = pltpu.PrefetchScalarGridSpec(
        num_scalar_prefetch=2,
        grid=(NB,),
        in_specs=[
            pl.BlockSpec((BLK, DPAD), lambda i, be, u: (_blk(i, be, u), 0)),
            pl.BlockSpec((1, DPAD, H), lambda i, be, u: (_exp(i, be, u), 0, 0)),
            pl.BlockSpec((1, 1, H), lambda i, be, u: (_exp(i, be, u), 0, 0)),
            pl.BlockSpec((1, H, H), lambda i, be, u: (_exp(i, be, u), 0, 0)),
            pl.BlockSpec((1, 1, H), lambda i, be, u: (_exp(i, be, u), 0, 0)),
            pl.BlockSpec((1, 1, H), lambda i, be, u: (_exp(i, be, u), 0, 0)),
            pl.BlockSpec((1, 1, 1), lambda i, be, u: (_exp(i, be, u), 0, 0)),
        ],
        out_specs=pl.BlockSpec((BLK, DPAD), lambda i, be, u: (_blk(i, be, u), 0)),
    )
    return pl.pallas_call(
        _mlp_body,
        grid_spec=grid_spec,
        out_shape=jax.ShapeDtypeStruct((P_CAP, DPAD), jnp.float32),
        interpret=interpret,
    )(be, used, xpad, w1p, b1, w2bf, b2, w3r, b3)


@functools.lru_cache(maxsize=1)
def _sc_kernels():
    mesh = plsc.VectorSubcoreMesh(core_axis_name="c", subcore_axis_name="s")

    @functools.partial(
        pl.kernel,
        mesh=mesh,
        out_type=jax.ShapeDtypeStruct((P_CAP, DPAD), jnp.float32),
        scratch_types=[
            pltpu.VMEM((CH, 128), jnp.int32),
            pltpu.VMEM((TPW, DPAD), jnp.float32),
            pltpu.SemaphoreType.DMA,
        ],
    )
    def _sc_scatter(dest_hbm, x_hbm, xpad_hbm, idx_v, rows_v, sem):
        wid = lax.axis_index("s") * 2 + lax.axis_index("c")
        pltpu.sync_copy(dest_hbm.at[wid], idx_v)
        pltpu.sync_copy(x_hbm.at[pl.ds(wid * TPW, TPW)], rows_v)
        for j in range(CH):
            pltpu.async_copy(rows_v.at[pl.ds(j * 128, 128)],
                             xpad_hbm.at[idx_v.at[j]], sem).wait()

    @functools.partial(
        pl.kernel,
        mesh=mesh,
        out_type=jax.ShapeDtypeStruct((NTOK, DPAD), jnp.float32),
        scratch_types=[
            pltpu.VMEM((CH, 128), jnp.int32),
            pltpu.VMEM((TPW, DPAD), jnp.float32),
            pltpu.SemaphoreType.DMA,
        ],
    )
    def _sc_gather(dest_hbm, ypad_hbm, out_hbm, idx_v, rows_v, sem):
        wid = lax.axis_index("s") * 2 + lax.axis_index("c")
        pltpu.sync_copy(dest_hbm.at[wid], idx_v)
        for j in range(CH):
            pltpu.async_copy(ypad_hbm.at[idx_v.at[j]],
                             rows_v.at[pl.ds(j * 128, 128)], sem).wait()
        pltpu.sync_copy(rows_v, out_hbm.at[pl.ds(wid * TPW, TPW)])

    return _sc_scatter, _sc_gather


def kernel(x, moneyness_bins, maturity_bins, W1, b1, W2, b2, W3, b3):
    mb = moneyness_bins.astype(jnp.int32).reshape(R, C)
    tb = maturity_bins.astype(jnp.int32).reshape(R, C)
    dest, be_full = _routing_call(mb, tb)
    be_flat = be_full.reshape(-1)
    be = be_flat[:NB]
    used = be_flat[NB:NB + 1]
    dest3 = dest.reshape(NW, CH, 128)

    sc_scatter, sc_gather = _sc_kernels()
    xp = jnp.pad(x, ((0, 0), (0, DPAD - DIN)))
    xpad = sc_scatter(dest3, xp)

    w1p = jnp.pad(W1, ((0, 0), (0, DPAD - DIN), (0, 0))).astype(jnp.bfloat16)
    ypad = _mlp_call(be, used, xpad, w1p, b1.reshape(NEXP, 1, H),
                     W2.astype(jnp.bfloat16), b2.reshape(NEXP, 1, H),
                     W3.reshape(NEXP, 1, H), b3.reshape(NEXP, 1, 1))

    outr = sc_gather(dest3, ypad)
    return outr[:, :1]


# speedup vs baseline: 5.7198x; 1.0720x over previous
import functools

import jax
import jax.numpy as jnp
from jax import lax
from jax.experimental import pallas as pl
from jax.experimental.pallas import tpu as pltpu
from jax.experimental.pallas import tpu_sc as plsc

NEXP = 16
NTOK = 16384
DIN = 23
DPAD = 128
H = 512
BLK = 512
P_CAP = NTOK + NEXP * BLK
NB = P_CAP // BLK
R = 128
C = 128
NW = 32
TPW = NTOK // NW
CH = TPW // 128


def _routing_body(mb_ref, tb_ref, x_ref, dest_ref, be_ref, xp_ref):
    xp_ref[:, :DIN] = x_ref[...]
    xp_ref[:, DIN:] = jnp.zeros((NTOK, DPAD - DIN), jnp.float32)

    flat = mb_ref[...] * 4 + tb_ref[...]
    ri = lax.broadcasted_iota(jnp.int32, (R, C), 0)
    ci = lax.broadcasted_iota(jnp.int32, (R, C), 1)
    upper = (ri <= ci).astype(jnp.float32)
    lower = (ci < ri).astype(jnp.float32)
    ones = jnp.ones((R, C), jnp.float32)

    rank = jnp.zeros((R, C), jnp.float32)
    counts = []
    for e in range(NEXP):
        m = (flat == e).astype(jnp.float32)
        incl = jnp.dot(m, upper, preferred_element_type=jnp.float32)
        col_pre = jnp.dot(lower, m, preferred_element_type=jnp.float32)
        row_pre = jnp.dot(col_pre, ones, preferred_element_type=jnp.float32)
        rank = rank + m * (incl - m + row_pre)
        counts.append(jnp.sum(m).astype(jnp.int32))

    offs = []
    cumnb = []
    acc = jnp.int32(0)
    for e in range(NEXP):
        nbe = (counts[e] + (BLK - 1)) // BLK
        offs.append(acc * BLK)
        acc = acc + nbe
        cumnb.append(acc)

    offvec = jnp.zeros((R, C), jnp.float32)
    for e in range(NEXP):
        m = (flat == e).astype(jnp.float32)
        offvec = offvec + m * offs[e].astype(jnp.float32)
    dest_ref[...] = (rank + offvec).astype(jnp.int32)

    k = (lax.broadcasted_iota(jnp.int32, (8, 128), 0) * 128
         + lax.broadcasted_iota(jnp.int32, (8, 128), 1))
    be = jnp.zeros((8, 128), jnp.int32)
    for e in range(NEXP):
        be = be + (k >= cumnb[e]).astype(jnp.int32)
    be = jnp.minimum(be, NEXP - 1)
    be_ref[...] = jnp.where(k == NB, cumnb[NEXP - 1], be)


def _routing_call(mb, tb, x, interpret=False):
    return pl.pallas_call(
        _routing_body,
        out_shape=(jax.ShapeDtypeStruct((R, C), jnp.int32),
                   jax.ShapeDtypeStruct((8, 128), jnp.int32),
                   jax.ShapeDtypeStruct((NTOK, DPAD), jnp.float32)),
        interpret=interpret,
    )(mb, tb, x)


def _mlp_body(be_ref, used_ref, x_ref, w1_ref, b1_ref, w2_ref, b2_ref,
              w3_ref, b3_ref, o_ref):
    @pl.when(pl.program_id(0) < used_ref[0])
    def _():
        x = x_ref[...]
        h = jnp.dot(x[:, :DIN].astype(jnp.bfloat16),
                    w1_ref[0].astype(jnp.bfloat16),
                    preferred_element_type=jnp.float32)
        h = jnp.maximum(h + b1_ref[0], 0.0)
        h2 = jnp.dot(h.astype(jnp.bfloat16), w2_ref[0].astype(jnp.bfloat16),
                     preferred_element_type=jnp.float32)
        h2 = jnp.maximum(h2 + b2_ref[0], 0.0)
        y = (jnp.sum(h2 * w3_ref[0], axis=1, keepdims=True)
             + b3_ref[0, 0, 0])
        o_ref[...] = jnp.broadcast_to(y, (BLK, DPAD))


def _mlp_call(be, used, xpad, w1p, b1, w2bf, b2, w3r, b3, interpret=False):
    def _blk(i, be, used):
        return jnp.minimum(i, used[0] - 1)

    def _exp(i, be, used):
        return be[jnp.minimum(i, used[0] - 1)]

    grid_spec = pltpu.PrefetchScalarGridSpec(
        num_scalar_prefetch=2,
        grid=(NB,),
        in_specs=[
            pl.BlockSpec((BLK, DPAD), lambda i, be, u: (_blk(i, be, u), 0)),
            pl.BlockSpec((1, DIN, H), lambda i, be, u: (_exp(i, be, u), 0, 0)),
            pl.BlockSpec((1, 1, H), lambda i, be, u: (_exp(i, be, u), 0, 0)),
            pl.BlockSpec((1, H, H), lambda i, be, u: (_exp(i, be, u), 0, 0)),
            pl.BlockSpec((1, 1, H), lambda i, be, u: (_exp(i, be, u), 0, 0)),
            pl.BlockSpec((1, 1, H), lambda i, be, u: (_exp(i, be, u), 0, 0)),
            pl.BlockSpec((1, 1, 1), lambda i, be, u: (_exp(i, be, u), 0, 0)),
        ],
        out_specs=pl.BlockSpec((BLK, DPAD), lambda i, be, u: (_blk(i, be, u), 0)),
    )
    return pl.pallas_call(
        _mlp_body,
        grid_spec=grid_spec,
        out_shape=jax.ShapeDtypeStruct((P_CAP, DPAD), jnp.float32),
        interpret=interpret,
    )(be, used, xpad, w1p, b1, w2bf, b2, w3r, b3)


@functools.lru_cache(maxsize=1)
def _sc_kernels():
    mesh = plsc.VectorSubcoreMesh(core_axis_name="c", subcore_axis_name="s")

    @functools.partial(
        pl.kernel,
        mesh=mesh,
        out_type=jax.ShapeDtypeStruct((P_CAP, DPAD), jnp.float32),
        scratch_types=[
            pltpu.VMEM((CH, 128), jnp.int32),
            pltpu.VMEM((TPW, DPAD), jnp.float32),
            pltpu.SemaphoreType.DMA,
        ],
    )
    def _sc_scatter(dest_hbm, x_hbm, xpad_hbm, idx_v, rows_v, sem):
        wid = lax.axis_index("s") * 2 + lax.axis_index("c")
        pltpu.sync_copy(dest_hbm.at[wid], idx_v)
        pltpu.sync_copy(x_hbm.at[pl.ds(wid * TPW, TPW)], rows_v)
        for j in range(CH):
            pltpu.async_copy(rows_v.at[pl.ds(j * 128, 128)],
                             xpad_hbm.at[idx_v.at[j]], sem).wait()

    @functools.partial(
        pl.kernel,
        mesh=mesh,
        out_type=jax.ShapeDtypeStruct((NTOK, DPAD), jnp.float32),
        scratch_types=[
            pltpu.VMEM((CH, 128), jnp.int32),
            pltpu.VMEM((TPW, DPAD), jnp.float32),
            pltpu.SemaphoreType.DMA,
        ],
    )
    def _sc_gather(dest_hbm, ypad_hbm, out_hbm, idx_v, rows_v, sem):
        wid = lax.axis_index("s") * 2 + lax.axis_index("c")
        pltpu.sync_copy(dest_hbm.at[wid], idx_v)
        for j in range(CH):
            pltpu.async_copy(ypad_hbm.at[idx_v.at[j]],
                             rows_v.at[pl.ds(j * 128, 128)], sem).wait()
        pltpu.sync_copy(rows_v, out_hbm.at[pl.ds(wid * TPW, TPW)])

    return _sc_scatter, _sc_gather


def kernel(x, moneyness_bins, maturity_bins, W1, b1, W2, b2, W3, b3):
    mb = moneyness_bins.astype(jnp.int32).reshape(R, C)
    tb = maturity_bins.astype(jnp.int32).reshape(R, C)
    dest, be_full, xp = _routing_call(mb, tb, x)
    be_flat = be_full.reshape(-1)
    be = be_flat[:NB]
    used = be_flat[NB:NB + 1]
    dest3 = dest.reshape(NW, CH, 128)

    sc_scatter, sc_gather = _sc_kernels()
    xpad = sc_scatter(dest3, xp)

    ypad = _mlp_call(be, used, xpad, W1, b1.reshape(NEXP, 1, H),
                     W2, b2.reshape(NEXP, 1, H),
                     W3.reshape(NEXP, 1, H), b3.reshape(NEXP, 1, 1))

    outr = sc_gather(dest3, ypad)
    return outr[:, :1]
